# baseline (speedup 1.0000x reference)
import functools

import jax
import jax.numpy as jnp
from jax import lax
from jax.experimental import pallas as pl
from jax.experimental.pallas import tpu as pltpu
from jax.experimental.pallas import tpu_sc as plsc

D_MODEL = 384
_SCALE = float(D_MODEL) ** 0.5
_LANES = 16

_NW = 32
_NS = 16
_CHUNK = 64


def _embed_body(idx_hbm, table_hbm, out_hbm, idx_v,
                g0, g1, w0, w1, spmem,
                gs0, gs1, wsem, csem, osem, *, n_chunks):
    gbufs, wbufs = (g0, g1), (w0, w1)
    gsems = (gs0, gs1)
    sid = lax.axis_index("s")
    wid = sid * 2 + lax.axis_index("c")
    base_row = wid * (n_chunks * _CHUNK)
    pltpu.sync_copy(idx_hbm.at[wid], idx_v)

    def gather_start(c, b):
        pltpu.make_async_copy(
            table_hbm.at[idx_v.at[c]], gbufs[b], gsems[b]).start()

    def gather_wait(b):
        pltpu.make_async_copy(table_hbm.at[idx_v.at[0]], gbufs[b],
                              gsems[b]).wait()

    def out_rows(c):
        return out_hbm.at[pl.ds(base_row + c * _CHUNK, _CHUNK)]

    def scale_chunk(b):
        def row_body(j, rcarry, gbuf=gbufs[b], wbuf=wbufs[b]):
            for i in range(D_MODEL // _LANES):
                sl = pl.ds(i * _LANES, _LANES)
                wbuf[j, sl] = gbuf[j, sl] * _SCALE
            return rcarry

        lax.fori_loop(0, _CHUNK, row_body, 0)

    gather_start(0, 0)
    gather_start(1, 1)

    def pass_body(p, carry):
        c0 = p * 2
        c1 = c0 + 1

        @pl.when(p >= 1)
        def _():
            pltpu.make_async_copy(wbufs[1], spmem.at[sid], csem).wait()
            pltpu.make_async_copy(spmem.at[sid], out_rows(c0 - 1),
                                  osem).start()

        gather_wait(0)
        scale_chunk(0)

        @pl.when(c0 + 2 < n_chunks)
        def _():
            gather_start(c0 + 2, 0)

        @pl.when(c0 >= 2)
        def _():
            pltpu.make_async_copy(wbufs[0], out_rows(0), wsem).wait()

        pltpu.make_async_copy(wbufs[0], out_rows(c0), wsem).start()

        gather_wait(1)
        scale_chunk(1)

        @pl.when(c1 + 2 < n_chunks)
        def _():
            gather_start(c1 + 2, 1)

        @pl.when(p >= 1)
        def _():
            pltpu.make_async_copy(spmem.at[sid], out_rows(0), osem).wait()

        pltpu.make_async_copy(wbufs[1], spmem.at[sid], csem).start()
        return carry

    lax.fori_loop(0, n_chunks // 2, pass_body, 0)
    pltpu.make_async_copy(wbufs[1], spmem.at[sid], csem).wait()
    pltpu.make_async_copy(spmem.at[sid], out_rows(n_chunks - 1), osem).start()
    pltpu.make_async_copy(wbufs[0], out_rows(0), wsem).wait()
    pltpu.make_async_copy(spmem.at[sid], out_rows(0), osem).wait()


def kernel(x, W):
    orig_shape = x.shape
    b_total = x.size
    assert b_total % (_NW * _CHUNK) == 0
    n_chunks = b_total // (_NW * _CHUNK)
    assert n_chunks % 2 == 0
    idx = x.reshape(_NW, n_chunks, _CHUNK).astype(jnp.int32)

    mesh = plsc.VectorSubcoreMesh(core_axis_name="c", subcore_axis_name="s")
    run = functools.partial(
        pl.kernel,
        mesh=mesh,
        out_type=jax.ShapeDtypeStruct((b_total, D_MODEL), jnp.float32),
        scratch_types=(
            [pltpu.VMEM((n_chunks, _CHUNK), jnp.int32)]
            + [pltpu.VMEM((_CHUNK, D_MODEL), jnp.float32)] * 4
            + [pltpu.VMEM_SHARED((_NS, _CHUNK, D_MODEL), jnp.float32)]
            + [pltpu.SemaphoreType.DMA] * 5
        ),
    )(functools.partial(_embed_body, n_chunks=n_chunks))
    out = run(idx, W)
    return out.reshape(*orig_shape, D_MODEL)
